# baseline (speedup 1.0000x reference)
import functools

import jax
import jax.numpy as jnp
from jax import lax
from jax.experimental import pallas as pl
from jax.experimental.pallas import tpu as pltpu
from jax.experimental.pallas import tpu_sc as plsc

N_ENT = 10000
N_REL = 8
D = 128
N_BASES = 4
N_EDGES = 160000
BATCH = 1024
N_NEG = 64
EPS = 1e-5

NC = 2
NS = 16
NW = NC * NS

CH = 128
EPW = 5120
E_PAD = NW * EPW
NCHUNK = EPW // CH
ACH = 128
AHALF = ACH // 2
TOT_CHUNKS = E_PAD // ACH
A0 = 40
A1 = (TOT_CHUNKS // NS) - A0
AMAX = max(A0, A1)
SLAB_PAD = AMAX
ANCHUNK = EPW // ACH
DUMMY = N_ENT
ROWS_PAD = 10112
RPS = ROWS_PAD // NS
DEGW = 128

BN = 1000
NB = N_ENT // BN


def _acc_body(idx_hbm, dst_hbm, h_hbm, acc_out, idx_v, dst_v, rows_v,
              acc_sh, gsem0, gsem1, gsem2, gsem3):
    c = lax.axis_index("c")
    s = lax.axis_index("s")
    wid = c * NS + s

    def fill_zero(i, carry):
        for t in range(2):
            for j in range(D // 16):
                rows_v[t, i, pl.ds(j * 16, 16)] = jnp.zeros((16,), jnp.float32)
        return carry
    lax.fori_loop(0, ACH, fill_zero, 0)

    base_r = s * RPS
    for k in range(RPS // ACH):
        pltpu.sync_copy(rows_v.at[0], acc_sh.at[pl.ds(base_r + k * ACH, ACH)])
    rem = RPS % ACH
    if rem:
        off = base_r + (RPS // ACH) * ACH
        pltpu.sync_copy(rows_v.at[0, pl.ds(0, rem)], acc_sh.at[pl.ds(off, rem)])
    plsc.subcore_barrier()

    n_my = jnp.where(c == 0, A0, A1)
    row0 = jnp.where(c == 0, s * A0, NS * A0 + s * A1)
    pltpu.sync_copy(idx_hbm.at[pl.ds(row0, AMAX)], idx_v)
    pltpu.sync_copy(dst_hbm.at[pl.ds(row0, AMAX)], dst_v)

    sems = ((gsem0, gsem1), (gsem2, gsem3))

    def fire_gather(g, b):
        for p in range(2):
            pltpu.async_copy(
                h_hbm.at[idx_v.at[g, pl.ds(p * AHALF, AHALF)]],
                rows_v.at[b, pl.ds(p * AHALF, AHALF)], sems[b][p])

    def wait_gather(g, b):
        for p in range(2):
            pltpu.make_async_copy(
                h_hbm.at[idx_v.at[g, pl.ds(p * AHALF, AHALF)]],
                rows_v.at[b, pl.ds(p * AHALF, AHALF)], sems[b][p]).wait()

    fire_gather(0, 0)

    def edge_pair(i, carry):
        g0 = 2 * i
        g1 = g0 + 1
        fire_gather(g1, 1)
        wait_gather(g0, 0)
        pltpu.sync_copy(rows_v.at[0], acc_sh.at[dst_v.at[g0]], add=True)

        @pl.when(g0 + 2 < n_my)
        def _():
            fire_gather(g0 + 2, 0)
        wait_gather(g1, 1)
        pltpu.sync_copy(rows_v.at[1], acc_sh.at[dst_v.at[g1]], add=True)
        return carry
    lax.fori_loop(0, n_my // 2, edge_pair, 0)
    plsc.subcore_barrier()

    pltpu.sync_copy(acc_sh.at[pl.ds(base_r, RPS)],
                    acc_out.at[c, pl.ds(base_r, RPS)])


_acc_pass = pl.kernel(
    _acc_body,
    out_type=jax.ShapeDtypeStruct((NC, ROWS_PAD, D), jnp.float32),
    mesh=plsc.VectorSubcoreMesh(core_axis_name="c", subcore_axis_name="s",
                                num_cores=NC, num_subcores=NS),
    scratch_types=[
        pltpu.VMEM((AMAX, ACH), jnp.int32),
        pltpu.VMEM((AMAX, ACH), jnp.int32),
        pltpu.VMEM((2, ACH, D), jnp.float32),
        pltpu.VMEM_SHARED((ROWS_PAD, D), jnp.float32),
        pltpu.SemaphoreType.DMA,
        pltpu.SemaphoreType.DMA,
        pltpu.SemaphoreType.DMA,
        pltpu.SemaphoreType.DMA,
    ],
    name="rgcn_acc_pass",
)


def _deg_body(dst_hbm, deg_out, dst_v, ones_v, deg_sh):
    c = lax.axis_index("c")
    s = lax.axis_index("s")
    wid = c * NS + s

    def fill(val):
        def body(i, carry):
            for j in range(DEGW // 16):
                ones_v[i, pl.ds(j * 16, 16)] = jnp.full((16,), val, jnp.float32)
            return carry
        lax.fori_loop(0, CH, body, 0)

    fill(0.0)
    base_r = s * RPS
    for k in range(RPS // 128):
        pltpu.sync_copy(ones_v, deg_sh.at[pl.ds(base_r + k * 128, 128)])
    rem = RPS % 128
    if rem:
        off = base_r + (RPS // 128) * 128
        pltpu.sync_copy(ones_v.at[pl.ds(0, rem)], deg_sh.at[pl.ds(off, rem)])
    plsc.subcore_barrier()

    fill(1.0)
    row0 = wid * NCHUNK
    pltpu.sync_copy(dst_hbm.at[pl.ds(row0, NCHUNK)], dst_v)

    def edge_chunk(g, carry):
        pltpu.sync_copy(ones_v, deg_sh.at[dst_v.at[g]], add=True)
        return carry
    lax.fori_loop(0, NCHUNK, edge_chunk, 0)
    plsc.subcore_barrier()

    pltpu.sync_copy(deg_sh.at[pl.ds(base_r, RPS)],
                    deg_out.at[c, pl.ds(base_r, RPS)])


_deg_pass = pl.kernel(
    _deg_body,
    out_type=jax.ShapeDtypeStruct((NC, ROWS_PAD, DEGW), jnp.float32),
    mesh=plsc.VectorSubcoreMesh(core_axis_name="c", subcore_axis_name="s",
                                num_cores=NC, num_subcores=NS),
    scratch_types=[
        pltpu.VMEM((NCHUNK, CH), jnp.int32),
        pltpu.VMEM((CH, DEGW), jnp.float32),
        pltpu.VMEM_SHARED((ROWS_PAD, DEGW), jnp.float32),
    ],
    name="rgcn_deg_pass",
)


HPW = BATCH // NW
NPW_CH = BATCH * N_NEG // (NW * CH)


def _gather_body(emb_hbm, rel_hbm, hidx_hbm, tidx_hbm, ridx_hbm, nidx_hbm,
                 head_out, tail_out, relv_out, neg_out,
                 idx_v, rows_v, nidx_v, nrows_v, gsem):
    c = lax.axis_index("c")
    s = lax.axis_index("s")
    wid = c * NS + s
    base = wid * HPW

    def small_gather(src_idx_hbm, table_hbm, out_hbm):
        pltpu.sync_copy(src_idx_hbm.at[pl.ds(base, HPW)], idx_v)
        pltpu.async_copy(table_hbm.at[idx_v], rows_v, gsem).wait()
        pltpu.sync_copy(rows_v, out_hbm.at[pl.ds(base, HPW)])

    small_gather(hidx_hbm, emb_hbm, head_out)
    small_gather(tidx_hbm, emb_hbm, tail_out)
    small_gather(ridx_hbm, rel_hbm, relv_out)

    nrow0 = wid * NPW_CH
    pltpu.sync_copy(nidx_hbm.at[pl.ds(nrow0, NPW_CH)], nidx_v)

    def neg_chunk(k, carry):
        pltpu.async_copy(emb_hbm.at[nidx_v.at[k]], nrows_v, gsem).wait()
        pltpu.sync_copy(nrows_v, neg_out.at[pl.ds((nrow0 + k) * CH, CH)])
        return carry
    lax.fori_loop(0, NPW_CH, neg_chunk, 0)


_gather_pass = pl.kernel(
    _gather_body,
    out_type=[
        jax.ShapeDtypeStruct((BATCH, D), jnp.float32),
        jax.ShapeDtypeStruct((BATCH, D), jnp.float32),
        jax.ShapeDtypeStruct((BATCH, D), jnp.float32),
        jax.ShapeDtypeStruct((BATCH * N_NEG, D), jnp.float32),
    ],
    mesh=plsc.VectorSubcoreMesh(core_axis_name="c", subcore_axis_name="s",
                                num_cores=NC, num_subcores=NS),
    scratch_types=[
        pltpu.VMEM((HPW,), jnp.int32),
        pltpu.VMEM((HPW, D), jnp.float32),
        pltpu.VMEM((NPW_CH, CH), jnp.int32),
        pltpu.VMEM((CH, D), jnp.float32),
        pltpu.SemaphoreType.DMA,
    ],
    name="rgcn_score_gather",
)


def _compose_w(bases, coeffs_ref, r):
    w = coeffs_ref[r, 0] * bases[0]
    for b in range(1, N_BASES):
        w = w + coeffs_ref[r, b] * bases[b]
    return w


def _bn_apply(pre, stats, g, b):
    mean = stats[0:1, :] * (1.0 / N_ENT)
    ex2 = stats[1:2, :] * (1.0 / N_ENT)
    var = ex2 - mean * mean
    inv = lax.rsqrt(var + EPS)
    return jnp.maximum((pre - mean) * inv * g[0:1, :] + b[0:1, :], 0.0)


def _rel_mm_body(emb_ref, bases_ref, coeffs_ref, h_ref):
    emb = emb_ref[...]
    for r in range(N_REL):
        w = _compose_w(bases_ref[...], coeffs_ref, r)
        h_ref[r] = jnp.dot(emb, w, preferred_element_type=jnp.float32)


def _rel_mm_bn_body(pre_ref, stats_ref, g_ref, b_ref, bases_ref, coeffs_ref, h_ref):
    emb = _bn_apply(pre_ref[...], stats_ref[...], g_ref[...], b_ref[...])
    for r in range(N_REL):
        w = _compose_w(bases_ref[...], coeffs_ref, r)
        h_ref[r] = jnp.dot(emb, w, preferred_element_type=jnp.float32)


def _rel_mm(emb, bases, coeffs):
    out = pl.pallas_call(
        _rel_mm_body,
        grid=(NB,),
        in_specs=[
            pl.BlockSpec((BN, D), lambda nb: (nb, 0)),
            pl.BlockSpec((N_BASES, D, D), lambda nb: (0, 0, 0)),
            pl.BlockSpec(memory_space=pltpu.SMEM),
        ],
        out_specs=pl.BlockSpec((N_REL, BN, D), lambda nb: (0, nb, 0)),
        out_shape=jax.ShapeDtypeStruct((N_REL, N_ENT, D), jnp.float32),
    )(emb, bases, coeffs)
    return out.reshape(N_REL * N_ENT, D)


def _rel_mm_bn(pre, stats, g, b, bases, coeffs):
    out = pl.pallas_call(
        _rel_mm_bn_body,
        grid=(NB,),
        in_specs=[
            pl.BlockSpec((BN, D), lambda nb: (nb, 0)),
            pl.BlockSpec((8, D), lambda nb: (0, 0)),
            pl.BlockSpec((1, D), lambda nb: (0, 0)),
            pl.BlockSpec((1, D), lambda nb: (0, 0)),
            pl.BlockSpec((N_BASES, D, D), lambda nb: (0, 0, 0)),
            pl.BlockSpec(memory_space=pltpu.SMEM),
        ],
        out_specs=pl.BlockSpec((N_REL, BN, D), lambda nb: (0, nb, 0)),
        out_shape=jax.ShapeDtypeStruct((N_REL, N_ENT, D), jnp.float32),
    )(pre, stats, g, b, bases, coeffs)
    return out.reshape(N_REL * N_ENT, D)


def _make_combine(with_bn):
    def body(*refs):
        if with_bn:
            (emb_ref, stats_in_ref, g_ref, b_ref, slw_ref, slb_ref,
             acc_ref, deg_ref, pre_ref, stats_ref) = refs
            emb = _bn_apply(emb_ref[...], stats_in_ref[...], g_ref[...], b_ref[...])
        else:
            (emb_ref, slw_ref, slb_ref, acc_ref, deg_ref,
             pre_ref, stats_ref) = refs
            emb = emb_ref[...]
        pre = jnp.dot(emb, slw_ref[...], preferred_element_type=jnp.float32)
        pre = (pre + slb_ref[0:1, :]
               + acc_ref[0].astype(jnp.float32)
               + acc_ref[1].astype(jnp.float32))
        deg = deg_ref[0, :, 0:1] + deg_ref[1, :, 0:1]
        pre = pre / jnp.maximum(deg, 1.0)
        pre_ref[...] = pre
        ssum = jnp.sum(pre, axis=0, keepdims=True)
        ssq = jnp.sum(pre * pre, axis=0, keepdims=True)
        blk = jnp.concatenate(
            [ssum, ssq, jnp.zeros((6, D), jnp.float32)], axis=0)
        nb = pl.program_id(0)

        @pl.when(nb == 0)
        def _():
            stats_ref[...] = blk

        @pl.when(nb != 0)
        def _():
            stats_ref[...] = stats_ref[...] + blk

    full = pl.BlockSpec((BN, D), lambda nb: (nb, 0))
    row = pl.BlockSpec((1, D), lambda nb: (0, 0))
    in_specs = [full]
    if with_bn:
        in_specs += [pl.BlockSpec((8, D), lambda nb: (0, 0)), row, row]
    in_specs += [
        pl.BlockSpec((D, D), lambda nb: (0, 0)),
        row,
        pl.BlockSpec((NC, BN, D), lambda nb: (0, nb, 0)),
        pl.BlockSpec((NC, BN, DEGW), lambda nb: (0, nb, 0)),
    ]

    def run(*args):
        return pl.pallas_call(
            body,
            grid=(NB,),
            in_specs=in_specs,
            out_specs=[
                pl.BlockSpec((BN, D), lambda nb: (nb, 0)),
                pl.BlockSpec((8, D), lambda nb: (0, 0)),
            ],
            out_shape=[
                jax.ShapeDtypeStruct((N_ENT, D), jnp.float32),
                jax.ShapeDtypeStruct((8, D), jnp.float32),
            ],
        )(*args)
    return run


_combine_plain = _make_combine(False)
_combine_bn = _make_combine(True)


def _bn_final_body(pre_ref, stats_ref, g_ref, b_ref, emb_ref):
    emb_ref[...] = _bn_apply(pre_ref[...], stats_ref[...], g_ref[...], b_ref[...])


def _bn_final(pre, stats, g, b):
    row = pl.BlockSpec((1, D), lambda nb: (0, 0))
    return pl.pallas_call(
        _bn_final_body,
        grid=(NB,),
        in_specs=[
            pl.BlockSpec((BN, D), lambda nb: (nb, 0)),
            pl.BlockSpec((8, D), lambda nb: (0, 0)),
            row, row,
        ],
        out_specs=pl.BlockSpec((BN, D), lambda nb: (nb, 0)),
        out_shape=jax.ShapeDtypeStruct((N_ENT, D), jnp.float32),
    )(pre, stats, g, b)


SB = 128


def _score_body(h_ref, r_ref, t_ref, n_ref, pos_ref, neg_ref):
    h = h_ref[...]
    r = r_ref[...]
    t = t_ref[...]
    d = h + r - t
    pos_ref[pl.program_id(0), :] = -jnp.sqrt(jnp.sum(d * d, axis=1))
    hr = (h + r)[:, None, :]
    nd = hr - n_ref[...]
    neg_ref[...] = -jnp.sqrt(jnp.sum(nd * nd, axis=2))


def _scores(head_e, rel_e, tail_e, neg_e3):
    blk = pl.BlockSpec((SB, D), lambda i: (i, 0))
    pos, neg = pl.pallas_call(
        _score_body,
        grid=(BATCH // SB,),
        in_specs=[
            blk, blk, blk,
            pl.BlockSpec((SB, N_NEG, D), lambda i: (i, 0, 0)),
        ],
        out_specs=[
            pl.BlockSpec((BATCH // SB, SB), lambda i: (0, 0)),
            pl.BlockSpec((SB, N_NEG), lambda i: (i, 0)),
        ],
        out_shape=[
            jax.ShapeDtypeStruct((BATCH // SB, SB), jnp.float32),
            jax.ShapeDtypeStruct((BATCH, N_NEG), jnp.float32),
        ],
    )(head_e, rel_e, tail_e, neg_e3)
    return pos.reshape(BATCH), neg


def kernel(head_idx, relation_idx, tail_idx, negative_idx, edge_index,
           edge_type, entity_table, relation_table, bases0, coeffs0, slw0,
           slb0, bn_g0, bn_b0, bases1, coeffs1, slw1, slb1, bn_g1, bn_b1):
    i32 = jnp.int32
    pad = E_PAD - N_EDGES
    spad = SLAB_PAD * ACH
    flat_idx = (edge_type.astype(i32) * N_ENT + edge_index[0].astype(i32))
    idx_flat = jnp.concatenate([flat_idx, jnp.zeros((pad + spad,), i32)])
    dst_flat = jnp.concatenate(
        [edge_index[1].astype(i32), jnp.full((pad + spad,), DUMMY, i32)])
    idx_a = idx_flat.reshape(-1, ACH)
    dst_a = dst_flat.reshape(-1, ACH)
    dst_p = dst_flat[:E_PAD].reshape(E_PAD // CH, CH)

    slb0r = slb0.reshape(1, D)
    slb1r = slb1.reshape(1, D)
    g0 = bn_g0.reshape(1, D)
    b0 = bn_b0.reshape(1, D)
    g1 = bn_g1.reshape(1, D)
    b1 = bn_b1.reshape(1, D)

    deg_raw = _deg_pass(dst_p)

    h0 = _rel_mm(entity_table, bases0, coeffs0)
    h0, deg_raw = lax.optimization_barrier((h0, deg_raw))
    deg = deg_raw[:, :N_ENT]
    acc0 = _acc_pass(idx_a, dst_a, h0)
    pre0, stats0 = _combine_plain(entity_table, slw0, slb0r,
                                  acc0[:, :N_ENT], deg)

    h1 = _rel_mm_bn(pre0, stats0, g0, b0, bases1, coeffs1)
    acc1 = _acc_pass(idx_a, dst_a, h1)
    pre1, stats1 = _combine_bn(pre0, stats0, g0, b0, slw1, slb1r,
                               acc1[:, :N_ENT], deg)

    emb2 = _bn_final(pre1, stats1, g1, b1)
    head_e, tail_e, rel_e, neg_e = _gather_pass(
        emb2, relation_table,
        head_idx.astype(i32), tail_idx.astype(i32), relation_idx.astype(i32),
        negative_idx.astype(i32).reshape(BATCH * N_NEG // CH, CH))
    return _scores(head_e, rel_e, tail_e,
                   neg_e.reshape(BATCH, N_NEG, D))


# speedup vs baseline: 1.0572x; 1.0572x over previous
import functools

import jax
import jax.numpy as jnp
from jax import lax
from jax.experimental import pallas as pl
from jax.experimental.pallas import tpu as pltpu
from jax.experimental.pallas import tpu_sc as plsc

N_ENT = 10000
N_REL = 8
D = 128
N_BASES = 4
N_EDGES = 160000
BATCH = 1024
N_NEG = 64
EPS = 1e-5

NC = 2
NS = 16
NW = NC * NS

CH = 128
EPW = 5120
E_PAD = NW * EPW
NCHUNK = EPW // CH
ACH = 128
AHALF = ACH // 2
ANCHUNK = EPW // ACH
DUMMY = N_ENT
ROWS_PAD = 10112
RPS = ROWS_PAD // NS
DEGW = 128

BN = 1000
NB = N_ENT // BN


def _acc_body(idx_hbm, dst_hbm, h_hbm, acc_out, idx_v, dst_v, rows_v,
              acc_sh, gsem0, gsem1, gsem2, gsem3):
    c = lax.axis_index("c")
    s = lax.axis_index("s")
    wid = c * NS + s

    def fill_zero(i, carry):
        for t in range(2):
            for j in range(D // 16):
                rows_v[t, i, pl.ds(j * 16, 16)] = jnp.zeros((16,), jnp.float32)
        return carry
    lax.fori_loop(0, ACH, fill_zero, 0)

    base_r = s * RPS
    for k in range(RPS // ACH):
        pltpu.sync_copy(rows_v.at[0], acc_sh.at[pl.ds(base_r + k * ACH, ACH)])
    rem = RPS % ACH
    if rem:
        off = base_r + (RPS // ACH) * ACH
        pltpu.sync_copy(rows_v.at[0, pl.ds(0, rem)], acc_sh.at[pl.ds(off, rem)])
    plsc.subcore_barrier()

    row0 = wid * ANCHUNK
    pltpu.sync_copy(idx_hbm.at[pl.ds(row0, ANCHUNK)], idx_v)
    pltpu.sync_copy(dst_hbm.at[pl.ds(row0, ANCHUNK)], dst_v)

    sems = ((gsem0, gsem1), (gsem2, gsem3))

    def fire_gather(g, b):
        for p in range(2):
            pltpu.async_copy(
                h_hbm.at[idx_v.at[g, pl.ds(p * AHALF, AHALF)]],
                rows_v.at[b, pl.ds(p * AHALF, AHALF)], sems[b][p])

    def wait_gather(g, b):
        for p in range(2):
            pltpu.make_async_copy(
                h_hbm.at[idx_v.at[g, pl.ds(p * AHALF, AHALF)]],
                rows_v.at[b, pl.ds(p * AHALF, AHALF)], sems[b][p]).wait()

    fire_gather(0, 0)

    def edge_pair(i, carry):
        g0 = 2 * i
        g1 = g0 + 1
        fire_gather(g1, 1)
        wait_gather(g0, 0)
        pltpu.sync_copy(rows_v.at[0], acc_sh.at[dst_v.at[g0]], add=True)

        @pl.when(g0 + 2 < ANCHUNK)
        def _():
            fire_gather(g0 + 2, 0)
        wait_gather(g1, 1)
        pltpu.sync_copy(rows_v.at[1], acc_sh.at[dst_v.at[g1]], add=True)
        return carry
    lax.fori_loop(0, ANCHUNK // 2, edge_pair, 0)
    plsc.subcore_barrier()

    pltpu.sync_copy(acc_sh.at[pl.ds(base_r, RPS)],
                    acc_out.at[c, pl.ds(base_r, RPS)])


_acc_pass = pl.kernel(
    _acc_body,
    out_type=jax.ShapeDtypeStruct((NC, ROWS_PAD, D), jnp.float32),
    mesh=plsc.VectorSubcoreMesh(core_axis_name="c", subcore_axis_name="s",
                                num_cores=NC, num_subcores=NS),
    scratch_types=[
        pltpu.VMEM((ANCHUNK, ACH), jnp.int32),
        pltpu.VMEM((ANCHUNK, ACH), jnp.int32),
        pltpu.VMEM((2, ACH, D), jnp.float32),
        pltpu.VMEM_SHARED((ROWS_PAD, D), jnp.float32),
        pltpu.SemaphoreType.DMA,
        pltpu.SemaphoreType.DMA,
        pltpu.SemaphoreType.DMA,
        pltpu.SemaphoreType.DMA,
    ],
    name="rgcn_acc_pass",
)


def _deg_body(dst_hbm, deg_out, dst_v, ones_v, deg_sh):
    c = lax.axis_index("c")
    s = lax.axis_index("s")
    wid = c * NS + s

    def fill(val):
        def body(i, carry):
            for j in range(DEGW // 16):
                ones_v[i, pl.ds(j * 16, 16)] = jnp.full((16,), val, jnp.float32)
            return carry
        lax.fori_loop(0, CH, body, 0)

    fill(0.0)
    base_r = s * RPS
    for k in range(RPS // 128):
        pltpu.sync_copy(ones_v, deg_sh.at[pl.ds(base_r + k * 128, 128)])
    rem = RPS % 128
    if rem:
        off = base_r + (RPS // 128) * 128
        pltpu.sync_copy(ones_v.at[pl.ds(0, rem)], deg_sh.at[pl.ds(off, rem)])
    plsc.subcore_barrier()

    fill(1.0)
    row0 = wid * NCHUNK
    pltpu.sync_copy(dst_hbm.at[pl.ds(row0, NCHUNK)], dst_v)

    def edge_chunk(g, carry):
        pltpu.sync_copy(ones_v, deg_sh.at[dst_v.at[g]], add=True)
        return carry
    lax.fori_loop(0, NCHUNK, edge_chunk, 0)
    plsc.subcore_barrier()

    pltpu.sync_copy(deg_sh.at[pl.ds(base_r, RPS)],
                    deg_out.at[c, pl.ds(base_r, RPS)])


_deg_pass = pl.kernel(
    _deg_body,
    out_type=jax.ShapeDtypeStruct((NC, ROWS_PAD, DEGW), jnp.float32),
    mesh=plsc.VectorSubcoreMesh(core_axis_name="c", subcore_axis_name="s",
                                num_cores=NC, num_subcores=NS),
    scratch_types=[
        pltpu.VMEM((NCHUNK, CH), jnp.int32),
        pltpu.VMEM((CH, DEGW), jnp.float32),
        pltpu.VMEM_SHARED((ROWS_PAD, DEGW), jnp.float32),
    ],
    name="rgcn_deg_pass",
)


HPW = BATCH // NW
NPW_CH = BATCH * N_NEG // (NW * CH)


def _gather_body(emb_hbm, rel_hbm, hidx_hbm, tidx_hbm, ridx_hbm, nidx_hbm,
                 head_out, tail_out, relv_out, neg_out,
                 idx_v, rows_v, nidx_v, nrows_v, gsem):
    c = lax.axis_index("c")
    s = lax.axis_index("s")
    wid = c * NS + s
    base = wid * HPW

    def small_gather(src_idx_hbm, table_hbm, out_hbm):
        pltpu.sync_copy(src_idx_hbm.at[pl.ds(base, HPW)], idx_v)
        pltpu.async_copy(table_hbm.at[idx_v], rows_v, gsem).wait()
        pltpu.sync_copy(rows_v, out_hbm.at[pl.ds(base, HPW)])

    small_gather(hidx_hbm, emb_hbm, head_out)
    small_gather(tidx_hbm, emb_hbm, tail_out)
    small_gather(ridx_hbm, rel_hbm, relv_out)

    nrow0 = wid * NPW_CH
    pltpu.sync_copy(nidx_hbm.at[pl.ds(nrow0, NPW_CH)], nidx_v)

    def neg_chunk(k, carry):
        pltpu.async_copy(emb_hbm.at[nidx_v.at[k]], nrows_v, gsem).wait()
        pltpu.sync_copy(nrows_v, neg_out.at[pl.ds((nrow0 + k) * CH, CH)])
        return carry
    lax.fori_loop(0, NPW_CH, neg_chunk, 0)


_gather_pass = pl.kernel(
    _gather_body,
    out_type=[
        jax.ShapeDtypeStruct((BATCH, D), jnp.float32),
        jax.ShapeDtypeStruct((BATCH, D), jnp.float32),
        jax.ShapeDtypeStruct((BATCH, D), jnp.float32),
        jax.ShapeDtypeStruct((BATCH * N_NEG, D), jnp.float32),
    ],
    mesh=plsc.VectorSubcoreMesh(core_axis_name="c", subcore_axis_name="s",
                                num_cores=NC, num_subcores=NS),
    scratch_types=[
        pltpu.VMEM((HPW,), jnp.int32),
        pltpu.VMEM((HPW, D), jnp.float32),
        pltpu.VMEM((NPW_CH, CH), jnp.int32),
        pltpu.VMEM((CH, D), jnp.float32),
        pltpu.SemaphoreType.DMA,
    ],
    name="rgcn_score_gather",
)


def _compose_w(bases, coeffs_ref, r):
    w = coeffs_ref[r, 0] * bases[0]
    for b in range(1, N_BASES):
        w = w + coeffs_ref[r, b] * bases[b]
    return w


def _bn_apply(pre, stats, g, b):
    mean = stats[0:1, :] * (1.0 / N_ENT)
    ex2 = stats[1:2, :] * (1.0 / N_ENT)
    var = ex2 - mean * mean
    inv = lax.rsqrt(var + EPS)
    return jnp.maximum((pre - mean) * inv * g[0:1, :] + b[0:1, :], 0.0)


def _rel_mm_body(emb_ref, bases_ref, coeffs_ref, h_ref):
    emb = emb_ref[...]
    for r in range(N_REL):
        w = _compose_w(bases_ref[...], coeffs_ref, r)
        h_ref[r] = jnp.dot(emb, w, preferred_element_type=jnp.float32)


def _rel_mm_bn_body(pre_ref, stats_ref, g_ref, b_ref, bases_ref, coeffs_ref, h_ref):
    emb = _bn_apply(pre_ref[...], stats_ref[...], g_ref[...], b_ref[...])
    for r in range(N_REL):
        w = _compose_w(bases_ref[...], coeffs_ref, r)
        h_ref[r] = jnp.dot(emb, w, preferred_element_type=jnp.float32)


def _rel_mm(emb, bases, coeffs):
    out = pl.pallas_call(
        _rel_mm_body,
        grid=(NB,),
        in_specs=[
            pl.BlockSpec((BN, D), lambda nb: (nb, 0)),
            pl.BlockSpec((N_BASES, D, D), lambda nb: (0, 0, 0)),
            pl.BlockSpec(memory_space=pltpu.SMEM),
        ],
        out_specs=pl.BlockSpec((N_REL, BN, D), lambda nb: (0, nb, 0)),
        out_shape=jax.ShapeDtypeStruct((N_REL, N_ENT, D), jnp.float32),
    )(emb, bases, coeffs)
    return out.reshape(N_REL * N_ENT, D)


def _rel_mm_bn(pre, stats, g, b, bases, coeffs):
    out = pl.pallas_call(
        _rel_mm_bn_body,
        grid=(NB,),
        in_specs=[
            pl.BlockSpec((BN, D), lambda nb: (nb, 0)),
            pl.BlockSpec((8, D), lambda nb: (0, 0)),
            pl.BlockSpec((1, D), lambda nb: (0, 0)),
            pl.BlockSpec((1, D), lambda nb: (0, 0)),
            pl.BlockSpec((N_BASES, D, D), lambda nb: (0, 0, 0)),
            pl.BlockSpec(memory_space=pltpu.SMEM),
        ],
        out_specs=pl.BlockSpec((N_REL, BN, D), lambda nb: (0, nb, 0)),
        out_shape=jax.ShapeDtypeStruct((N_REL, N_ENT, D), jnp.float32),
    )(pre, stats, g, b, bases, coeffs)
    return out.reshape(N_REL * N_ENT, D)


def _make_combine(with_bn):
    def body(*refs):
        if with_bn:
            (emb_ref, stats_in_ref, g_ref, b_ref, slw_ref, slb_ref,
             acc_ref, deg_ref, pre_ref, stats_ref) = refs
            emb = _bn_apply(emb_ref[...], stats_in_ref[...], g_ref[...], b_ref[...])
        else:
            (emb_ref, slw_ref, slb_ref, acc_ref, deg_ref,
             pre_ref, stats_ref) = refs
            emb = emb_ref[...]
        pre = jnp.dot(emb, slw_ref[...], preferred_element_type=jnp.float32)
        pre = (pre + slb_ref[0:1, :]
               + acc_ref[0].astype(jnp.float32)
               + acc_ref[1].astype(jnp.float32))
        deg = deg_ref[0, :, 0:1] + deg_ref[1, :, 0:1]
        pre = pre / jnp.maximum(deg, 1.0)
        pre_ref[...] = pre
        ssum = jnp.sum(pre, axis=0, keepdims=True)
        ssq = jnp.sum(pre * pre, axis=0, keepdims=True)
        blk = jnp.concatenate(
            [ssum, ssq, jnp.zeros((6, D), jnp.float32)], axis=0)
        nb = pl.program_id(0)

        @pl.when(nb == 0)
        def _():
            stats_ref[...] = blk

        @pl.when(nb != 0)
        def _():
            stats_ref[...] = stats_ref[...] + blk

    full = pl.BlockSpec((BN, D), lambda nb: (nb, 0))
    row = pl.BlockSpec((1, D), lambda nb: (0, 0))
    in_specs = [full]
    if with_bn:
        in_specs += [pl.BlockSpec((8, D), lambda nb: (0, 0)), row, row]
    in_specs += [
        pl.BlockSpec((D, D), lambda nb: (0, 0)),
        row,
        pl.BlockSpec((NC, BN, D), lambda nb: (0, nb, 0)),
        pl.BlockSpec((NC, BN, DEGW), lambda nb: (0, nb, 0)),
    ]

    def run(*args):
        return pl.pallas_call(
            body,
            grid=(NB,),
            in_specs=in_specs,
            out_specs=[
                pl.BlockSpec((BN, D), lambda nb: (nb, 0)),
                pl.BlockSpec((8, D), lambda nb: (0, 0)),
            ],
            out_shape=[
                jax.ShapeDtypeStruct((N_ENT, D), jnp.float32),
                jax.ShapeDtypeStruct((8, D), jnp.float32),
            ],
        )(*args)
    return run


_combine_plain = _make_combine(False)
_combine_bn = _make_combine(True)


def _bn_final_body(pre_ref, stats_ref, g_ref, b_ref, emb_ref):
    emb_ref[...] = _bn_apply(pre_ref[...], stats_ref[...], g_ref[...], b_ref[...])


def _bn_final(pre, stats, g, b):
    row = pl.BlockSpec((1, D), lambda nb: (0, 0))
    return pl.pallas_call(
        _bn_final_body,
        grid=(NB,),
        in_specs=[
            pl.BlockSpec((BN, D), lambda nb: (nb, 0)),
            pl.BlockSpec((8, D), lambda nb: (0, 0)),
            row, row,
        ],
        out_specs=pl.BlockSpec((BN, D), lambda nb: (nb, 0)),
        out_shape=jax.ShapeDtypeStruct((N_ENT, D), jnp.float32),
    )(pre, stats, g, b)


SB = 128


def _score_body(h_ref, r_ref, t_ref, n_ref, pos_ref, neg_ref):
    h = h_ref[...]
    r = r_ref[...]
    t = t_ref[...]
    d = h + r - t
    pos_ref[pl.program_id(0), :] = -jnp.sqrt(jnp.sum(d * d, axis=1))
    hr = (h + r)[:, None, :]
    nd = hr - n_ref[...]
    neg_ref[...] = -jnp.sqrt(jnp.sum(nd * nd, axis=2))


def _scores(head_e, rel_e, tail_e, neg_e3):
    blk = pl.BlockSpec((SB, D), lambda i: (i, 0))
    pos, neg = pl.pallas_call(
        _score_body,
        grid=(BATCH // SB,),
        in_specs=[
            blk, blk, blk,
            pl.BlockSpec((SB, N_NEG, D), lambda i: (i, 0, 0)),
        ],
        out_specs=[
            pl.BlockSpec((BATCH // SB, SB), lambda i: (0, 0)),
            pl.BlockSpec((SB, N_NEG), lambda i: (i, 0)),
        ],
        out_shape=[
            jax.ShapeDtypeStruct((BATCH // SB, SB), jnp.float32),
            jax.ShapeDtypeStruct((BATCH, N_NEG), jnp.float32),
        ],
    )(head_e, rel_e, tail_e, neg_e3)
    return pos.reshape(BATCH), neg


def kernel(head_idx, relation_idx, tail_idx, negative_idx, edge_index,
           edge_type, entity_table, relation_table, bases0, coeffs0, slw0,
           slb0, bn_g0, bn_b0, bases1, coeffs1, slw1, slb1, bn_g1, bn_b1):
    i32 = jnp.int32
    pad = E_PAD - N_EDGES
    flat_idx = (edge_type.astype(i32) * N_ENT + edge_index[0].astype(i32))
    idx_flat = jnp.concatenate([flat_idx, jnp.zeros((pad,), i32)])
    dst_flat = jnp.concatenate(
        [edge_index[1].astype(i32), jnp.full((pad,), DUMMY, i32)])
    idx_a = idx_flat.reshape(-1, ACH)
    dst_a = dst_flat.reshape(-1, ACH)
    dst_p = dst_flat.reshape(E_PAD // CH, CH)

    slb0r = slb0.reshape(1, D)
    slb1r = slb1.reshape(1, D)
    g0 = bn_g0.reshape(1, D)
    b0 = bn_b0.reshape(1, D)
    g1 = bn_g1.reshape(1, D)
    b1 = bn_b1.reshape(1, D)

    deg_raw = _deg_pass(dst_p)

    h0 = _rel_mm(entity_table, bases0, coeffs0)
    h0, deg_raw = lax.optimization_barrier((h0, deg_raw))
    deg = deg_raw[:, :N_ENT]
    acc0 = _acc_pass(idx_a, dst_a, h0)
    pre0, stats0 = _combine_plain(entity_table, slw0, slb0r,
                                  acc0[:, :N_ENT], deg)

    h1 = _rel_mm_bn(pre0, stats0, g0, b0, bases1, coeffs1)
    acc1 = _acc_pass(idx_a, dst_a, h1)
    pre1, stats1 = _combine_bn(pre0, stats0, g0, b0, slw1, slb1r,
                               acc1[:, :N_ENT], deg)

    emb2 = _bn_final(pre1, stats1, g1, b1)
    head_e, tail_e, rel_e, neg_e = _gather_pass(
        emb2, relation_table,
        head_idx.astype(i32), tail_idx.astype(i32), relation_idx.astype(i32),
        negative_idx.astype(i32).reshape(BATCH * N_NEG // CH, CH))
    return _scores(head_e, rel_e, tail_e,
                   neg_e.reshape(BATCH, N_NEG, D))
